# baseline (speedup 1.0000x reference)
import functools

import jax
import jax.numpy as jnp
from jax import lax
from jax.experimental import pallas as pl
from jax.experimental.pallas import tpu as pltpu
from jax.experimental.pallas import tpu_sc as plsc

_IDX_CHUNK = 128


@functools.lru_cache(maxsize=None)
def _build(B: int, V: int, D: int):
  info = plsc.get_sparse_core_info()
  nc, ns = info.num_cores, info.num_subcores
  nw = nc * ns
  assert B % nw == 0
  b_per_w = B // nw
  n_chunks = -(-b_per_w // _IDX_CHUNK)
  mesh = plsc.VectorSubcoreMesh(core_axis_name="c", subcore_axis_name="s")

  @functools.partial(
      pl.kernel,
      mesh=mesh,
      out_type=jax.ShapeDtypeStruct((B, D), jnp.float32),
      scratch_types=[
          pltpu.VMEM((b_per_w,), jnp.int32),
          pltpu.VMEM((b_per_w, D), jnp.float32),
          pltpu.SemaphoreType.DMA,
      ],
      compiler_params=pltpu.CompilerParams(use_tc_tiling_on_sc=False),
  )
  def gather_kernel(idx_hbm, table_hbm, out_hbm, idx_v, rows_v, sem):
    wid = lax.axis_index("s") * nc + lax.axis_index("c")
    base = wid * b_per_w
    pltpu.sync_copy(idx_hbm.at[pl.ds(base, b_per_w)], idx_v)
    copies = []
    for j in range(n_chunks):
      lo = j * _IDX_CHUNK
      sz = min(_IDX_CHUNK, b_per_w - lo)
      copies.append(
          pltpu.async_copy(
              table_hbm.at[idx_v.at[pl.ds(lo, sz)]],
              rows_v.at[pl.ds(lo, sz)],
              sem,
          )
      )
    for c in copies:
      c.wait()
    pltpu.sync_copy(rows_v, out_hbm.at[pl.ds(base, b_per_w)])

  return gather_kernel


def kernel(movie_id, movie_table):
  (B,) = movie_id.shape
  V, D = movie_table.shape
  idx = movie_id.astype(jnp.int32)
  return _build(B, V, D)(idx, movie_table)


# speedup vs baseline: 3.8518x; 3.8518x over previous
import functools

import jax
import jax.numpy as jnp
from jax import lax
from jax.experimental import pallas as pl
from jax.experimental.pallas import tpu as pltpu
from jax.experimental.pallas import tpu_sc as plsc

_LANES = 16
_SUBL = 8
_TMIN = 128
_ROUND = 64


@functools.lru_cache(maxsize=None)
def _build(B: int, V: int, D: int):
  info = plsc.get_sparse_core_info()
  nw = info.num_cores * info.num_subcores
  d_tiles = D // _SUBL
  wpd = nw // d_tiles
  b_per_w = B // wpd
  n_rounds = b_per_w // _ROUND
  assert n_rounds * _ROUND == b_per_w and b_per_w % _TMIN == 0
  mesh = plsc.VectorSubcoreMesh(core_axis_name="c", subcore_axis_name="s")

  @functools.partial(
      pl.kernel,
      mesh=mesh,
      out_type=jax.ShapeDtypeStruct((D, B), jnp.float32),
      scratch_types=[
          pltpu.VMEM((b_per_w,), jnp.int32),
          pltpu.VMEM((_ROUND, _SUBL, _TMIN), jnp.float32),
          pltpu.VMEM((_SUBL, b_per_w), jnp.float32),
          pltpu.SemaphoreType.DMA,
      ],
      compiler_params=pltpu.CompilerParams(needs_layout_passes=False),
  )
  def gather_kernel(idx_hbm, table_t_hbm, out_t_hbm, idx_v, stage_v,
                    band_v, sem):
    wid = lax.axis_index("s") * info.num_cores + lax.axis_index("c")
    ti = wid // wpd
    d0 = pl.multiple_of(ti * _SUBL, _SUBL)
    gbase = (wid % wpd) * b_per_w
    pltpu.sync_copy(idx_hbm.at[pl.ds(gbase, b_per_w)], idx_v)

    def per_round(r):
      for m in range(_ROUND // _LANES):
        vi = idx_v[pl.ds(r * _ROUND + m * _LANES, _LANES)]
        vcol = vi >> 7
        for j in range(_LANES):
          col = pl.multiple_of(vcol[j] * _TMIN, _TMIN)
          pltpu.async_copy(
              table_t_hbm.at[pl.ds(d0, _SUBL), pl.ds(col, _TMIN)],
              stage_v.at[m * _LANES + j],
              sem,
          )
      for i in range(_ROUND):
        pltpu.make_async_copy(
            table_t_hbm.at[pl.ds(0, _SUBL), pl.ds(0, _TMIN)],
            stage_v.at[i],
            sem,
        ).wait()
      for m in range(_ROUND // _LANES):
        vi = idx_v[pl.ds(r * _ROUND + m * _LANES, _LANES)]
        lane = vi & (_TMIN - 1)
        row = lax.iota(jnp.int32, _LANES) + m * _LANES
        for s in range(_SUBL):
          vals = plsc.load_gather(
              stage_v, [row, jnp.full((_LANES,), s, jnp.int32), lane]
          )
          band_v[s, pl.ds(r * _ROUND + m * _LANES, _LANES)] = vals

    pl.loop(0, n_rounds)(per_round)

    pltpu.sync_copy(
        band_v,
        out_t_hbm.at[pl.ds(d0, _SUBL), pl.ds(gbase, b_per_w)],
    )

  return gather_kernel


def kernel(movie_id, movie_table):
  (B,) = movie_id.shape
  V, D = movie_table.shape
  idx = movie_id.astype(jnp.int32)
  out_t = _build(B, V, D)(idx, movie_table.T)
  return out_t.T


# speedup vs baseline: 4.7405x; 1.2307x over previous
import functools

import jax
import jax.numpy as jnp
from jax import lax
from jax.experimental import pallas as pl
from jax.experimental.pallas import tpu as pltpu
from jax.experimental.pallas import tpu_sc as plsc

_LANES = 16
_SUBL = 8
_TMIN = 128
_ROUND = 32


@functools.lru_cache(maxsize=None)
def _build(B: int, V: int, D: int):
  info = plsc.get_sparse_core_info()
  nw = info.num_cores * info.num_subcores
  d_tiles = D // _SUBL
  wpd = nw // d_tiles
  b_per_w = B // wpd
  n_rounds = b_per_w // _ROUND
  assert n_rounds * _ROUND == b_per_w and b_per_w % _TMIN == 0
  mesh = plsc.VectorSubcoreMesh(core_axis_name="c", subcore_axis_name="s")

  @functools.partial(
      pl.kernel,
      mesh=mesh,
      out_type=jax.ShapeDtypeStruct((D, B), jnp.float32),
      scratch_types=[
          pltpu.VMEM((b_per_w,), jnp.int32),
          pltpu.VMEM((2, _ROUND, _SUBL, _TMIN), jnp.float32),
          pltpu.VMEM((_SUBL, b_per_w), jnp.float32),
          pltpu.SemaphoreType.DMA,
      ],
      compiler_params=pltpu.CompilerParams(needs_layout_passes=False),
  )
  def gather_kernel(idx_hbm, table_t_hbm, out_t_hbm, idx_v, stage_v,
                    band_v, sem):
    wid = lax.axis_index("s") * info.num_cores + lax.axis_index("c")
    ti = wid // wpd
    d0 = pl.multiple_of(ti * _SUBL, _SUBL)
    gbase = (wid % wpd) * b_per_w
    pltpu.sync_copy(idx_hbm.at[pl.ds(gbase, b_per_w)], idx_v)

    def fire(r, buf):
      for m in range(_ROUND // _LANES):
        vi = idx_v[pl.ds(r * _ROUND + m * _LANES, _LANES)]
        vcol = vi >> 7
        for j in range(_LANES):
          col = pl.multiple_of(vcol[j] * _TMIN, _TMIN)
          pltpu.async_copy(
              table_t_hbm.at[pl.ds(d0, _SUBL), pl.ds(col, _TMIN)],
              stage_v.at[buf, m * _LANES + j],
              sem,
          )

    def drain(buf):
      for i in range(_ROUND):
        pltpu.make_async_copy(
            table_t_hbm.at[pl.ds(0, _SUBL), pl.ds(0, _TMIN)],
            stage_v.at[buf, i],
            sem,
        ).wait()

    def extract(r, buf):
      buf_ix = jnp.full((_LANES,), 0, jnp.int32) + buf
      for m in range(_ROUND // _LANES):
        vi = idx_v[pl.ds(r * _ROUND + m * _LANES, _LANES)]
        lane = vi & (_TMIN - 1)
        row = lax.iota(jnp.int32, _LANES) + m * _LANES
        for s in range(_SUBL):
          vals = plsc.load_gather(
              stage_v, [buf_ix, row, jnp.full((_LANES,), s, jnp.int32), lane]
          )
          band_v[s, pl.ds(r * _ROUND + m * _LANES, _LANES)] = vals

    def step(r):
      fire(r, r % 2)

      @pl.when(r > 0)
      def _():
        drain((r - 1) % 2)
        extract(r - 1, (r - 1) % 2)

    pl.loop(0, n_rounds)(step)
    drain((n_rounds - 1) % 2)
    extract(n_rounds - 1, (n_rounds - 1) % 2)

    pltpu.sync_copy(
        band_v,
        out_t_hbm.at[pl.ds(d0, _SUBL), pl.ds(gbase, b_per_w)],
    )

  return gather_kernel


def kernel(movie_id, movie_table):
  (B,) = movie_id.shape
  V, D = movie_table.shape
  idx = movie_id.astype(jnp.int32)
  out_t = _build(B, V, D)(idx, movie_table.T)
  return out_t.T
